# baseline (device time: 10980 ns/iter reference)
import jax
import jax.numpy as jnp
from jax import lax
from jax.experimental import pallas as pl
from jax.experimental.pallas import tpu as pltpu

C = 8


def kernel(x):
    _, m, n = x.shape
    half = n // 2
    rows = m // C

    def body(
        x_ref,
        out_hbm,
        qstage,
        qrecv,
        sscale,
        rscale,
        obuf,
        qsend_sems,
        qrecv_sems,
        ssend_sem,
        srecv_sem,
        out_sems,
    ):
        my_x = lax.axis_index("x")
        my_y = lax.axis_index("y")
        other_y = 1 - my_y
        col0 = my_y * half
        scol0 = other_y * half

        barrier_sem = pltpu.get_barrier_semaphore()
        pl.semaphore_signal(
            barrier_sem,
            inc=1,
            device_id=(my_x, other_y),
            device_id_type=pl.DeviceIdType.MESH,
        )

        def scale_rdma():
            return pltpu.make_async_remote_copy(
                src_ref=sscale,
                dst_ref=rscale,
                send_sem=ssend_sem,
                recv_sem=srecv_sem,
                device_id=(my_x, other_y),
                device_id_type=pl.DeviceIdType.MESH,
            )

        def chunk_rdma(c):
            return pltpu.make_async_remote_copy(
                src_ref=qstage.at[c],
                dst_ref=qrecv.at[c],
                send_sem=qsend_sems.at[c],
                recv_sem=qrecv_sems.at[c],
                device_id=(my_x, other_y),
                device_id_type=pl.DeviceIdType.MESH,
            )

        def out_dma(c):
            return pltpu.make_async_copy(
                obuf.at[c],
                out_hbm.at[pl.ds(c * rows, rows), :],
                out_sems.at[c],
            )

        col_ids = lax.broadcasted_iota(jnp.int32, (8, 128), 1)
        scale_tile = jnp.zeros((8, 128), jnp.float32)
        for c in range(C):
            v = x_ref[0, pl.ds(c * rows, rows), pl.ds(scol0, half)]
            m_abs = jnp.max(jnp.abs(v)) + 1e-30
            scale_tile = jnp.where(col_ids == c, m_abs / 127.0, scale_tile)
            qstage[c] = jnp.clip(
                jnp.round(v * (127.0 / m_abs)), -127.0, 127.0
            ).astype(jnp.int8)
            if c == 0:
                pl.semaphore_wait(barrier_sem, 1)
            chunk_rdma(c).start()
        sscale[...] = scale_tile
        scale_rdma().start()

        scale_rdma().wait_recv()
        rtile = rscale[...]
        for c in range(C):
            chunk_rdma(c).wait_recv()
            rs = jnp.max(jnp.where(col_ids == c, rtile, 0.0))
            obuf[c] = (
                x_ref[0, pl.ds(c * rows, rows), pl.ds(col0, half)]
                + qrecv[c].astype(jnp.float32) * rs
            ).astype(jnp.bfloat16)
            out_dma(c).start()

        scale_rdma().wait_send()
        for c in range(C):
            out_dma(c).wait()
            chunk_rdma(c).wait_send()

    return pl.pallas_call(
        body,
        out_shape=jax.ShapeDtypeStruct((m, half), jnp.bfloat16),
        in_specs=[pl.BlockSpec(memory_space=pltpu.VMEM)],
        out_specs=pl.BlockSpec(memory_space=pl.ANY),
        scratch_shapes=[
            pltpu.VMEM((C, rows, half), jnp.int8),
            pltpu.VMEM((C, rows, half), jnp.int8),
            pltpu.VMEM((8, 128), jnp.float32),
            pltpu.VMEM((8, 128), jnp.float32),
            pltpu.VMEM((C, rows, half), jnp.bfloat16),
            pltpu.SemaphoreType.DMA((C,)),
            pltpu.SemaphoreType.DMA((C,)),
            pltpu.SemaphoreType.DMA,
            pltpu.SemaphoreType.DMA,
            pltpu.SemaphoreType.DMA((C,)),
        ],
        compiler_params=pltpu.CompilerParams(collective_id=0),
    )(x)


# device time: 9474 ns/iter; 1.1590x vs baseline; 1.1590x over previous
import jax
import jax.numpy as jnp
from jax import lax
from jax.experimental import pallas as pl
from jax.experimental.pallas import tpu as pltpu

C = 2


def kernel(x):
    _, m, n = x.shape
    half = n // 2
    rows = m // C

    def body(
        x_ref,
        out_ref,
        qstage,
        qrecv,
        scale_send,
        scale_recv,
        send_sems,
        recv_sems,
        ssend_sem,
        srecv_sem,
    ):
        my_x = lax.axis_index("x")
        my_y = lax.axis_index("y")
        other_y = 1 - my_y
        col0 = my_y * half
        scol0 = other_y * half

        m_abs = jnp.max(jnp.abs(x_ref[0, :, pl.ds(scol0, half)])) + 1e-30
        scale_send[...] = jnp.full((8, 128), m_abs / 127.0, jnp.float32)
        inv = 127.0 / m_abs
        for c in range(C):
            v = x_ref[0, pl.ds(c * rows, rows), pl.ds(scol0, half)]
            qstage[c] = jnp.clip(
                jnp.round(v * inv), -127.0, 127.0
            ).astype(jnp.int8)

        barrier_sem = pltpu.get_barrier_semaphore()
        pl.semaphore_signal(
            barrier_sem,
            inc=1,
            device_id=(my_x, other_y),
            device_id_type=pl.DeviceIdType.MESH,
        )
        pl.semaphore_wait(barrier_sem, 1)

        def scale_rdma():
            return pltpu.make_async_remote_copy(
                src_ref=scale_send,
                dst_ref=scale_recv,
                send_sem=ssend_sem,
                recv_sem=srecv_sem,
                device_id=(my_x, other_y),
                device_id_type=pl.DeviceIdType.MESH,
            )

        def chunk_rdma(c):
            return pltpu.make_async_remote_copy(
                src_ref=qstage.at[c],
                dst_ref=qrecv.at[c],
                send_sem=send_sems.at[c],
                recv_sem=recv_sems.at[c],
                device_id=(my_x, other_y),
                device_id_type=pl.DeviceIdType.MESH,
            )

        scale_rdma().start()
        for c in range(C):
            chunk_rdma(c).start()

        scale_rdma().wait_recv()
        rs = jnp.max(scale_recv[...])
        for c in range(C):
            rdma = chunk_rdma(c)
            rdma.wait_recv()
            local = x_ref[0, pl.ds(c * rows, rows), pl.ds(col0, half)]
            out_ref[pl.ds(c * rows, rows), :] = (
                local + qrecv[c].astype(jnp.float32) * rs
            ).astype(jnp.bfloat16)

        scale_rdma().wait_send()
        for c in range(C):
            chunk_rdma(c).wait_send()

    return pl.pallas_call(
        body,
        out_shape=jax.ShapeDtypeStruct((m, half), jnp.bfloat16),
        in_specs=[pl.BlockSpec(memory_space=pltpu.VMEM)],
        out_specs=pl.BlockSpec(memory_space=pltpu.VMEM),
        scratch_shapes=[
            pltpu.VMEM((C, rows, half), jnp.int8),
            pltpu.VMEM((C, rows, half), jnp.int8),
            pltpu.VMEM((8, 128), jnp.float32),
            pltpu.VMEM((8, 128), jnp.float32),
            pltpu.SemaphoreType.DMA((C,)),
            pltpu.SemaphoreType.DMA((C,)),
            pltpu.SemaphoreType.DMA,
            pltpu.SemaphoreType.DMA,
        ],
        compiler_params=pltpu.CompilerParams(collective_id=0),
    )(x)


# device time: 9390 ns/iter; 1.1693x vs baseline; 1.0089x over previous
import jax
import jax.numpy as jnp
from jax import lax
from jax.experimental import pallas as pl
from jax.experimental.pallas import tpu as pltpu

C = 4


def kernel(x):
    _, m, n = x.shape
    half = n // 2
    rows = m // C

    def body(
        x_ref,
        out_ref,
        qstage,
        qrecv,
        scale_send,
        scale_recv,
        send_sems,
        recv_sems,
        ssend_sem,
        srecv_sem,
    ):
        my_x = lax.axis_index("x")
        my_y = lax.axis_index("y")
        other_y = 1 - my_y
        col0 = my_y * half
        scol0 = other_y * half

        barrier_sem = pltpu.get_barrier_semaphore()
        pl.semaphore_signal(
            barrier_sem,
            inc=1,
            device_id=(my_x, other_y),
            device_id_type=pl.DeviceIdType.MESH,
        )

        m_abs = jnp.max(jnp.abs(x_ref[0, :, pl.ds(scol0, half)])) + 1e-30
        scale_send[...] = jnp.full((8, 128), m_abs / 127.0, jnp.float32)
        inv = 127.0 / m_abs
        for c in range(C):
            v = x_ref[0, pl.ds(c * rows, rows), pl.ds(scol0, half)]
            qstage[c] = jnp.clip(
                jnp.round(v * inv), -127.0, 127.0
            ).astype(jnp.int8)

        pl.semaphore_wait(barrier_sem, 1)

        def scale_rdma():
            return pltpu.make_async_remote_copy(
                src_ref=scale_send,
                dst_ref=scale_recv,
                send_sem=ssend_sem,
                recv_sem=srecv_sem,
                device_id=(my_x, other_y),
                device_id_type=pl.DeviceIdType.MESH,
            )

        def chunk_rdma(c):
            return pltpu.make_async_remote_copy(
                src_ref=qstage.at[c],
                dst_ref=qrecv.at[c],
                send_sem=send_sems.at[c],
                recv_sem=recv_sems.at[c],
                device_id=(my_x, other_y),
                device_id_type=pl.DeviceIdType.MESH,
            )

        scale_rdma().start()
        for c in range(C):
            chunk_rdma(c).start()

        scale_rdma().wait_recv()
        rs = jnp.max(scale_recv[...])
        for c in range(C):
            rdma = chunk_rdma(c)
            rdma.wait_recv()
            local = x_ref[0, pl.ds(c * rows, rows), pl.ds(col0, half)]
            out_ref[pl.ds(c * rows, rows), :] = (
                local + qrecv[c].astype(jnp.float32) * rs
            ).astype(jnp.bfloat16)

        scale_rdma().wait_send()
        for c in range(C):
            chunk_rdma(c).wait_send()

    return pl.pallas_call(
        body,
        out_shape=jax.ShapeDtypeStruct((m, half), jnp.bfloat16),
        in_specs=[pl.BlockSpec(memory_space=pltpu.VMEM)],
        out_specs=pl.BlockSpec(memory_space=pltpu.VMEM),
        scratch_shapes=[
            pltpu.VMEM((C, rows, half), jnp.int8),
            pltpu.VMEM((C, rows, half), jnp.int8),
            pltpu.VMEM((8, 128), jnp.float32),
            pltpu.VMEM((8, 128), jnp.float32),
            pltpu.SemaphoreType.DMA((C,)),
            pltpu.SemaphoreType.DMA((C,)),
            pltpu.SemaphoreType.DMA,
            pltpu.SemaphoreType.DMA,
        ],
        compiler_params=pltpu.CompilerParams(collective_id=0),
    )(x)
